# Initial kernel scaffold; baseline (speedup 1.0000x reference)
import jax
import jax.numpy as jnp
from jax.experimental import pallas as pl


def kernel(warped_xyz, warped_points, f2_xyz, f2_points, params):
    raise NotImplementedError("write your pallas kernel here")


# speedup vs baseline: 9.5867x; 9.5867x over previous
import functools

import jax
import jax.numpy as jnp
from jax import lax
from jax.experimental import pallas as pl
from jax.experimental.pallas import tpu as pltpu
from jax.experimental.pallas import tpu_sc as plsc

_B, _N, _C = 2, 4096, 64
_KQ, _KC = 16, 8
_EPS = 1e-5
_ROW = 80
_T = 256
_CH = 128

_HI = jax.lax.Precision.HIGHEST


def _dot(a, b):
    return jnp.dot(a, b, precision=_HI, preferred_element_type=jnp.float32)


def _bmm3(x3, w):
    return (x3[:, 0:1] * w[0:1, :] + x3[:, 1:2] * w[1:2, :]
            + x3[:, 2:3] * w[2:3, :])


def _knn_body(k, q_ref, kt_ref, o_ref):
    b = pl.program_id(0)
    q = q_ref[0]
    q0, q1, q2 = q[:, 0:1], q[:, 1:2], q[:, 2:3]
    k0 = kt_ref[0, 0:1, :]
    k1 = kt_ref[0, 1:2, :]
    k2 = kt_ref[0, 2:3, :]
    e = q0 * k0 + q1 * k1 + q2 * k2
    n1 = q0 * q0 + q1 * q1 + q2 * q2
    n2 = k0 * k0 + k1 * k1 + k2 * k2
    d = -2.0 * e + n1 + n2
    iota = lax.broadcasted_iota(jnp.int32, (_T, _N), 1)
    big = jnp.int32(2**31 - 1)
    cols = []
    for _ in range(k):
        m = jnp.min(d, axis=1, keepdims=True)
        cand = jnp.where(d == m, iota, big)
        idx = jnp.min(cand, axis=1, keepdims=True)
        cols.append(idx)
        d = jnp.where(iota == idx, jnp.float32(jnp.inf), d)
    o_ref[0] = jnp.concatenate(cols, axis=1) + b * _N


def _knn(queries, keys_t, k):
    return pl.pallas_call(
        functools.partial(_knn_body, k),
        grid=(_B, _N // _T),
        in_specs=[
            pl.BlockSpec((1, _T, 3), lambda b, i: (b, i, 0)),
            pl.BlockSpec((1, 3, _N), lambda b, i: (b, 0, 0)),
        ],
        out_specs=pl.BlockSpec((1, _T, k), lambda b, i: (b, i, 0)),
        out_shape=jax.ShapeDtypeStruct((_B, _N, k), jnp.int32),
        compiler_params=pltpu.CompilerParams(
            dimension_semantics=("parallel", "parallel")),
    )(queries, keys_t)


def _sc_gather(table, idx_flat):
    M = idx_flat.shape[0]
    info = plsc.get_sparse_core_info()
    nc, ns = info.num_cores, info.num_subcores
    nw = nc * ns
    per_w = M // nw
    n_chunks = per_w // _CH
    mesh = plsc.VectorSubcoreMesh(core_axis_name="c", subcore_axis_name="s")

    @functools.partial(
        pl.kernel, mesh=mesh,
        out_type=jax.ShapeDtypeStruct((M, _ROW), jnp.float32),
        compiler_params=pltpu.CompilerParams(use_tc_tiling_on_sc=False),
        scratch_types=[
            pltpu.VMEM((_CH,), jnp.int32),
            pltpu.VMEM((_CH, _ROW), jnp.float32),
            pltpu.SemaphoreType.DMA,
        ],
    )
    def gk(table_hbm, idx_hbm, out_hbm, idx_v, rows_v, sem):
        wid = lax.axis_index("s") * nc + lax.axis_index("c")
        base = wid * per_w

        def body(i, carry):
            off = base + i * _CH
            pltpu.sync_copy(idx_hbm.at[pl.ds(off, _CH)], idx_v)
            pltpu.async_copy(table_hbm.at[idx_v], rows_v, sem).wait()
            pltpu.sync_copy(rows_v, out_hbm.at[pl.ds(off, _CH)])
            return carry

        lax.fori_loop(0, n_chunks, body, 0)

    return gk(table, idx_flat)


def _mlp1_body(wx_ref, wp_ref, g_ref,
               w1wx_ref, w1qx_ref, w1eu_ref, w1wp_ref, w1qp_ref, b1_ref,
               w2_ref, b2_ref, w3_ref, b3_ref,
               wewx_ref, weqx_ref, weeu_ref, be_ref,
               wm1e_ref, wm1x_ref, bm1_ref, wm2_ref, bm2_ref,
               o_ref):
    wx = wx_ref[0]
    wp = wp_ref[0]
    g = g_ref[0]
    base1 = (_bmm3(wx, w1wx_ref[...]) + _dot(wp, w1wp_ref[...])
             + b1_ref[...])
    ew = _bmm3(wx, wewx_ref[...]) + be_ref[...]
    logits, xs = [], []
    for k in range(_KQ):
        off = _ROW * k
        qx = g[:, off:off + 3]
        qp = g[:, off + 3:off + 67]
        diff = qx - wx
        euc = jnp.sqrt(diff[:, 0:1] * diff[:, 0:1]
                       + diff[:, 1:2] * diff[:, 1:2]
                       + diff[:, 2:3] * diff[:, 2:3] + 1e-20)
        h = base1 + _bmm3(qx, w1qx_ref[...]) + euc * w1eu_ref[...] \
            + _dot(qp, w1qp_ref[...])
        h = jnp.maximum(h, 0.0)
        h = jnp.maximum(_dot(h, w2_ref[...]) + b2_ref[...], 0.0)
        x = jnp.maximum(_dot(h, w3_ref[...]) + b3_ref[...], 0.0)
        e = jnp.maximum(ew + _bmm3(qx, weqx_ref[...])
                        + euc * weeu_ref[...], 0.0)
        l = _dot(e, wm1e_ref[...]) + _dot(x, wm1x_ref[...]) + bm1_ref[...]
        l = jnp.maximum(l, 0.0)
        l = jnp.maximum(_dot(l, wm2_ref[...]) + bm2_ref[...], 0.0)
        logits.append(l)
        xs.append(x)
    m = logits[0]
    for l in logits[1:]:
        m = jnp.maximum(m, l)
    ssum = jnp.zeros_like(m)
    acc = jnp.zeros_like(m)
    for l, x in zip(logits, xs):
        w = jnp.exp(l - m)
        ssum = ssum + w
        acc = acc + w * x
    o_ref[0] = acc / ssum


def _mlp1(wxyz, wpts, g1, weights):
    full = [pl.BlockSpec(w.shape, lambda b, i: (0,) * w.ndim) for w in weights]
    return pl.pallas_call(
        _mlp1_body,
        grid=(_B, _N // _T),
        in_specs=[
            pl.BlockSpec((1, _T, 3), lambda b, i: (b, i, 0)),
            pl.BlockSpec((1, _T, _C), lambda b, i: (b, i, 0)),
            pl.BlockSpec((1, _T, _KQ * _ROW), lambda b, i: (b, i, 0)),
        ] + full,
        out_specs=pl.BlockSpec((1, _T, _C), lambda b, i: (b, i, 0)),
        out_shape=jax.ShapeDtypeStruct((_B, _N, _C), jnp.float32),
        compiler_params=pltpu.CompilerParams(
            dimension_semantics=("parallel", "parallel")),
    )(wxyz, wpts, g1, *weights)


def _mlp2_body(wx_ref, wp_ref, g_ref,
               wewx_ref, weqx_ref, weeu_ref, be_ref,
               wm1e_ref, wm1wp_ref, wm1cp_ref, bm1_ref,
               wm2_ref, bm2_ref,
               o_ref):
    wx = wx_ref[0]
    wp = wp_ref[0]
    g = g_ref[0]
    ew = _bmm3(wx, wewx_ref[...]) + be_ref[...]
    basem = _dot(wp, wm1wp_ref[...]) + bm1_ref[...]
    logits, vals = [], []
    for k in range(_KC):
        off = _ROW * k
        cx = g[:, off:off + 3]
        cp = g[:, off + 3:off + 67]
        diff = cx - wx
        euc = jnp.sqrt(diff[:, 0:1] * diff[:, 0:1]
                       + diff[:, 1:2] * diff[:, 1:2]
                       + diff[:, 2:3] * diff[:, 2:3] + 1e-20)
        enc = jnp.maximum(ew + _bmm3(cx, weqx_ref[...])
                          + euc * weeu_ref[...], 0.0)
        l = basem + _dot(enc, wm1e_ref[...]) + _dot(cp, wm1cp_ref[...])
        l = jnp.maximum(l, 0.0)
        l = jnp.maximum(_dot(l, wm2_ref[...]) + bm2_ref[...], 0.0)
        logits.append(l)
        vals.append(cp)
    m = logits[0]
    for l in logits[1:]:
        m = jnp.maximum(m, l)
    ssum = jnp.zeros_like(m)
    acc = jnp.zeros_like(m)
    for l, v in zip(logits, vals):
        w = jnp.exp(l - m)
        ssum = ssum + w
        acc = acc + w * v
    o_ref[0] = acc / ssum


def _mlp2(wxyz, wpts, g2, weights):
    full = [pl.BlockSpec(w.shape, lambda b, i: (0,) * w.ndim) for w in weights]
    return pl.pallas_call(
        _mlp2_body,
        grid=(_B, _N // _T),
        in_specs=[
            pl.BlockSpec((1, _T, 3), lambda b, i: (b, i, 0)),
            pl.BlockSpec((1, _T, _C), lambda b, i: (b, i, 0)),
            pl.BlockSpec((1, _T, _KC * _ROW), lambda b, i: (b, i, 0)),
        ] + full,
        out_specs=pl.BlockSpec((1, _T, _C), lambda b, i: (b, i, 0)),
        out_shape=jax.ShapeDtypeStruct((_B, _N, _C), jnp.float32),
        compiler_params=pltpu.CompilerParams(
            dimension_semantics=("parallel", "parallel")),
    )(wxyz, wpts, g2, *weights)


def _fold(p):
    s = p["gamma"] / jnp.sqrt(p["var"] + _EPS)
    wt = (p["W"] * s[:, None]).T
    bt = ((p["b"] - p["mean"]) * s + p["beta"])[None, :]
    return wt, bt


def kernel(warped_xyz, warped_points, f2_xyz, f2_points, params):
    w1, b1 = _fold(params["mlp1"][0])
    w2, b2 = _fold(params["mlp1"][1])
    w3, b3 = _fold(params["mlp1"][2])
    we, be = _fold(params["pi_encoding"])
    wm1, bm1 = _fold(params["mlp2"][0])
    wm2, bm2 = _fold(params["mlp2"][1])
    wE, bE = _fold(params["pc_encoding"])
    wq1, bq1 = _fold(params["mlp2_2"][0])
    wq2, bq2 = _fold(params["mlp2_2"][1])

    w1_stack = [
        w1[0:3] - w1[6:9], w1[3:6] + w1[6:9], w1[9:10], w1[10:74], w1[74:138],
        b1, w2, b2, w3, b3,
        we[0:3] - we[6:9], we[3:6] + we[6:9], we[9:10], be,
        wm1[0:64], wm1[64:128], bm1, wm2, bm2,
    ]
    w2_stack = [
        wE[0:3] - wE[6:9], wE[3:6] + wE[6:9], wE[9:10], bE,
        wq1[0:64], wq1[64:128], wq1[128:192], bq1, wq2, bq2,
    ]

    f2_t = f2_xyz.transpose(0, 2, 1)
    w_t = warped_xyz.transpose(0, 2, 1)
    idx_q = _knn(warped_xyz, f2_t, _KQ)
    idx_c = _knn(warped_xyz, w_t, _KC)

    pad = jnp.zeros((_B, _N, _ROW - 3 - _C), jnp.float32)
    table1 = jnp.concatenate([f2_xyz, f2_points, pad], axis=-1)
    table1 = table1.reshape(_B * _N, _ROW)
    g1 = _sc_gather(table1, idx_q.reshape(-1))
    g1 = g1.reshape(_B, _N, _KQ * _ROW)

    pi_feat = _mlp1(warped_xyz, warped_points, g1, w1_stack)

    table2 = jnp.concatenate([warped_xyz, pi_feat, pad], axis=-1)
    table2 = table2.reshape(_B * _N, _ROW)
    g2 = _sc_gather(table2, idx_c.reshape(-1))
    g2 = g2.reshape(_B, _N, _KC * _ROW)

    return _mlp2(warped_xyz, warped_points, g2, w2_stack)


# speedup vs baseline: 15.6840x; 1.6360x over previous
import functools

import jax
import jax.numpy as jnp
from jax import lax
from jax.experimental import pallas as pl
from jax.experimental.pallas import tpu as pltpu
from jax.experimental.pallas import tpu_sc as plsc

_B, _N, _C = 2, 4096, 64
_KQ, _KC = 16, 8
_EPS = 1e-5
_ROW = 80
_T = 256
_CH = 128


def _dot(a, b):
    return jnp.dot(a.astype(jnp.bfloat16), b.astype(jnp.bfloat16),
                   preferred_element_type=jnp.float32)


def _knn_body(k, q_ref, kt_ref, o_ref):
    b = pl.program_id(0)
    q = q_ref[0]
    q0, q1, q2 = q[:, 0:1], q[:, 1:2], q[:, 2:3]
    k0 = kt_ref[0, 0:1, :]
    k1 = kt_ref[0, 1:2, :]
    k2 = kt_ref[0, 2:3, :]
    e = q0 * k0 + q1 * k1 + q2 * k2
    n1 = q0 * q0 + q1 * q1 + q2 * q2
    n2 = k0 * k0 + k1 * k1 + k2 * k2
    d = -2.0 * e + n1 + n2
    iota = lax.broadcasted_iota(jnp.int32, (_T, _N), 1)
    big = jnp.int32(2**31 - 1)
    cols = []
    for _ in range(k):
        m = jnp.min(d, axis=1, keepdims=True)
        cand = jnp.where(d == m, iota, big)
        idx = jnp.min(cand, axis=1, keepdims=True)
        cols.append(idx)
        d = jnp.where(iota == idx, jnp.float32(jnp.inf), d)
    o_ref[0] = jnp.concatenate(cols, axis=1) + b * _N


def _knn(queries, keys_t, k):
    return pl.pallas_call(
        functools.partial(_knn_body, k),
        grid=(_B, _N // _T),
        in_specs=[
            pl.BlockSpec((1, _T, 3), lambda b, i: (b, i, 0)),
            pl.BlockSpec((1, 3, _N), lambda b, i: (b, 0, 0)),
        ],
        out_specs=pl.BlockSpec((1, _T, k), lambda b, i: (b, i, 0)),
        out_shape=jax.ShapeDtypeStruct((_B, _N, k), jnp.int32),
        compiler_params=pltpu.CompilerParams(
            dimension_semantics=("parallel", "parallel")),
    )(queries, keys_t)


def _sc_gather(table, idx_flat):
    M = idx_flat.shape[0]
    info = plsc.get_sparse_core_info()
    nc, ns = info.num_cores, info.num_subcores
    nw = nc * ns
    per_w = M // nw
    n_chunks = per_w // _CH
    mesh = plsc.VectorSubcoreMesh(core_axis_name="c", subcore_axis_name="s")

    @functools.partial(
        pl.kernel, mesh=mesh,
        out_type=jax.ShapeDtypeStruct((M, _ROW), jnp.float32),
        compiler_params=pltpu.CompilerParams(use_tc_tiling_on_sc=False),
        scratch_types=[
            pltpu.VMEM((_CH,), jnp.int32),
            pltpu.VMEM((_CH, _ROW), jnp.float32),
            pltpu.SemaphoreType.DMA,
        ],
    )
    def gk(table_hbm, idx_hbm, out_hbm, idx_v, rows_v, sem):
        wid = lax.axis_index("s") * nc + lax.axis_index("c")
        base = wid * per_w

        def body(i, carry):
            off = base + i * _CH
            pltpu.sync_copy(idx_hbm.at[pl.ds(off, _CH)], idx_v)
            pltpu.async_copy(table_hbm.at[idx_v], rows_v, sem).wait()
            pltpu.sync_copy(rows_v, out_hbm.at[pl.ds(off, _CH)])
            return carry

        lax.fori_loop(0, n_chunks, body, 0)

    return gk(table, idx_flat)


def _mlp1_body(wx_ref, wp_ref, g_ref,
               wpxyz_ref, w1wp_ref, b1_ref, be_ref,
               wnxyz_ref, wneuc_ref, w1qp_ref,
               w2_ref, b2_ref, w3_ref, b3_ref,
               wm1e_ref, wm1x_ref, bm1_ref, wm2_ref, bm2_ref,
               o_ref):
    R = _KQ * _T
    wx = wx_ref[0]
    wp = wp_ref[0]
    gg = g_ref[0].reshape(R, _ROW)
    pxyz = _dot(wx, wpxyz_ref[...])
    base1 = pxyz[:, 0:128] + _dot(wp, w1wp_ref[...]) + b1_ref[...]
    ew = pxyz[:, 128:192] + be_ref[...]
    base1_t = jnp.broadcast_to(base1[None], (_KQ, _T, 128)).reshape(R, 128)
    ew_t = jnp.broadcast_to(ew[None], (_KQ, _T, _C)).reshape(R, _C)
    wx_t = jnp.broadcast_to(wx[None], (_KQ, _T, 3)).reshape(R, 3)
    qx = gg[:, 0:3]
    qp = gg[:, 3:67]
    diff = qx - wx_t
    euc = jnp.sqrt(diff[:, 0:1] * diff[:, 0:1]
                   + diff[:, 1:2] * diff[:, 1:2]
                   + diff[:, 2:3] * diff[:, 2:3] + 1e-20)
    nxyz = _dot(qx, wnxyz_ref[...]) + _dot(euc, wneuc_ref[...])
    h = jnp.maximum(base1_t + nxyz[:, 0:128] + _dot(qp, w1qp_ref[...]), 0.0)
    h = jnp.maximum(_dot(h, w2_ref[...]) + b2_ref[...], 0.0)
    x = jnp.maximum(_dot(h, w3_ref[...]) + b3_ref[...], 0.0)
    e = jnp.maximum(ew_t + nxyz[:, 128:192], 0.0)
    l = _dot(e, wm1e_ref[...]) + _dot(x, wm1x_ref[...]) + bm1_ref[...]
    l = jnp.maximum(l, 0.0)
    l = jnp.maximum(_dot(l, wm2_ref[...]) + bm2_ref[...], 0.0)
    logits = [l[_T * k:_T * (k + 1)] for k in range(_KQ)]
    xs = [x[_T * k:_T * (k + 1)] for k in range(_KQ)]
    m = logits[0]
    for lk in logits[1:]:
        m = jnp.maximum(m, lk)
    ssum = jnp.zeros_like(m)
    acc = jnp.zeros_like(m)
    for lk, xk in zip(logits, xs):
        w = jnp.exp(lk - m)
        ssum = ssum + w
        acc = acc + w * xk
    o_ref[0] = acc / ssum


def _mlp1(wxyz, wpts, g1, weights):
    full = [pl.BlockSpec(w.shape, lambda b, i: (0,) * w.ndim) for w in weights]
    return pl.pallas_call(
        _mlp1_body,
        grid=(_B, _N // _T),
        in_specs=[
            pl.BlockSpec((1, _T, 3), lambda b, i: (b, i, 0)),
            pl.BlockSpec((1, _T, _C), lambda b, i: (b, i, 0)),
            pl.BlockSpec((1, _KQ, _T, _ROW), lambda b, i: (b, 0, i, 0)),
        ] + full,
        out_specs=pl.BlockSpec((1, _T, _C), lambda b, i: (b, i, 0)),
        out_shape=jax.ShapeDtypeStruct((_B, _N, _C), jnp.float32),
        compiler_params=pltpu.CompilerParams(
            dimension_semantics=("parallel", "parallel")),
    )(wxyz, wpts, g1, *weights)


def _mlp2_body(wx_ref, wp_ref, g_ref,
               wpxyz_ref, be_ref, wnxyz_ref, wneuc_ref,
               wm1e_ref, wm1wp_ref, wm1cp_ref, bm1_ref,
               wm2_ref, bm2_ref,
               o_ref):
    R = _KC * _T
    wx = wx_ref[0]
    wp = wp_ref[0]
    gg = g_ref[0].reshape(R, _ROW)
    ew = _dot(wx, wpxyz_ref[...]) + be_ref[...]
    basem = _dot(wp, wm1wp_ref[...]) + bm1_ref[...]
    ew_t = jnp.broadcast_to(ew[None], (_KC, _T, _C)).reshape(R, _C)
    basem_t = jnp.broadcast_to(basem[None], (_KC, _T, 128)).reshape(R, 128)
    wx_t = jnp.broadcast_to(wx[None], (_KC, _T, 3)).reshape(R, 3)
    cx = gg[:, 0:3]
    cp = gg[:, 3:67]
    diff = cx - wx_t
    euc = jnp.sqrt(diff[:, 0:1] * diff[:, 0:1]
                   + diff[:, 1:2] * diff[:, 1:2]
                   + diff[:, 2:3] * diff[:, 2:3] + 1e-20)
    enc = jnp.maximum(ew_t + _dot(cx, wnxyz_ref[...])
                      + _dot(euc, wneuc_ref[...]), 0.0)
    l = basem_t + _dot(enc, wm1e_ref[...]) + _dot(cp, wm1cp_ref[...])
    l = jnp.maximum(l, 0.0)
    l = jnp.maximum(_dot(l, wm2_ref[...]) + bm2_ref[...], 0.0)
    logits = [l[_T * k:_T * (k + 1)] for k in range(_KC)]
    vals = [cp[_T * k:_T * (k + 1)] for k in range(_KC)]
    m = logits[0]
    for lk in logits[1:]:
        m = jnp.maximum(m, lk)
    ssum = jnp.zeros_like(m)
    acc = jnp.zeros_like(m)
    for lk, v in zip(logits, vals):
        w = jnp.exp(lk - m)
        ssum = ssum + w
        acc = acc + w * v
    o_ref[0] = acc / ssum


def _mlp2(wxyz, wpts, g2, weights):
    full = [pl.BlockSpec(w.shape, lambda b, i: (0,) * w.ndim) for w in weights]
    return pl.pallas_call(
        _mlp2_body,
        grid=(_B, _N // _T),
        in_specs=[
            pl.BlockSpec((1, _T, 3), lambda b, i: (b, i, 0)),
            pl.BlockSpec((1, _T, _C), lambda b, i: (b, i, 0)),
            pl.BlockSpec((1, _KC, _T, _ROW), lambda b, i: (b, 0, i, 0)),
        ] + full,
        out_specs=pl.BlockSpec((1, _T, _C), lambda b, i: (b, i, 0)),
        out_shape=jax.ShapeDtypeStruct((_B, _N, _C), jnp.float32),
        compiler_params=pltpu.CompilerParams(
            dimension_semantics=("parallel", "parallel")),
    )(wxyz, wpts, g2, *weights)


def _fold(p):
    s = p["gamma"] / jnp.sqrt(p["var"] + _EPS)
    wt = (p["W"] * s[:, None]).T
    bt = ((p["b"] - p["mean"]) * s + p["beta"])[None, :]
    return wt, bt


def kernel(warped_xyz, warped_points, f2_xyz, f2_points, params):
    w1, b1 = _fold(params["mlp1"][0])
    w2, b2 = _fold(params["mlp1"][1])
    w3, b3 = _fold(params["mlp1"][2])
    we, be = _fold(params["pi_encoding"])
    wm1, bm1 = _fold(params["mlp2"][0])
    wm2, bm2 = _fold(params["mlp2"][1])
    wE, bE = _fold(params["pc_encoding"])
    wq1, bq1 = _fold(params["mlp2_2"][0])
    wq2, bq2 = _fold(params["mlp2_2"][1])

    w1_stack = [
        jnp.concatenate([w1[0:3] - w1[6:9], we[0:3] - we[6:9]], axis=1),
        w1[10:74], b1, be,
        jnp.concatenate([w1[3:6] + w1[6:9], we[3:6] + we[6:9]], axis=1),
        jnp.concatenate([w1[9:10], we[9:10]], axis=1),
        w1[74:138],
        w2, b2, w3, b3,
        wm1[0:64], wm1[64:128], bm1, wm2, bm2,
    ]
    w2_stack = [
        wE[0:3] - wE[6:9], bE, wE[3:6] + wE[6:9], wE[9:10],
        wq1[0:64], wq1[64:128], wq1[128:192], bq1, wq2, bq2,
    ]

    f2_t = f2_xyz.transpose(0, 2, 1)
    w_t = warped_xyz.transpose(0, 2, 1)
    idx_q = _knn(warped_xyz, f2_t, _KQ)

    pad = jnp.zeros((_B, _N, _ROW - 3 - _C), jnp.float32)
    table1 = jnp.concatenate([f2_xyz, f2_points, pad], axis=-1)
    table1 = table1.reshape(_B * _N, _ROW)
    g1 = _sc_gather(table1, idx_q.transpose(0, 2, 1).reshape(-1))
    g1 = g1.reshape(_B, _KQ, _N, _ROW)

    idx_c = _knn(warped_xyz, w_t, _KC)

    pi_feat = _mlp1(warped_xyz, warped_points, g1, w1_stack)

    table2 = jnp.concatenate([warped_xyz, pi_feat, pad], axis=-1)
    table2 = table2.reshape(_B * _N, _ROW)
    g2 = _sc_gather(table2, idx_c.transpose(0, 2, 1).reshape(-1))
    g2 = g2.reshape(_B, _KC, _N, _ROW)

    return _mlp2(warped_xyz, warped_points, g2, w2_stack)
